# baseline (speedup 1.0000x reference)
import jax
import jax.numpy as jnp
import numpy as np
from jax.experimental import pallas as pl
from jax.experimental.pallas import tpu as pltpu

E = 8
C1 = 96
C2 = 192
B = 32
H = W = 56
OH = OW = 28
NPOS = OH * OW
KTAPS = 9
AROWS = KTAPS * C1
LN_EPS = 1e-6
IPS = 2
GB = 8


def _gsum_kernel(x_ref, o_ref):
    o_ref[...] = x_ref[...].sum(axis=2)


def _moe_kernel(gs_ref, planes_ref, wmat_ref, b_ref, g_ref, beta_ref,
                wg_ref, cm_ref, out_ref, loss_ref, acc_ref):
    i = pl.program_id(0)
    nstep = pl.num_programs(0)
    cm = cm_ref[...]

    def shift_r(v):
        return jnp.concatenate(
            [jnp.zeros((C1, OW), jnp.bfloat16), v[:, :NPOS - OW]], axis=1)

    def shift_c(v):
        s = jnp.concatenate(
            [jnp.zeros((C1, 1), jnp.bfloat16), v[:, :NPOS - 1]], axis=1)
        return s * cm

    @pl.when(i == 0)
    def _():
        acc_ref[...] = jnp.zeros_like(acc_ref)

    eidx = jax.lax.broadcasted_iota(jnp.int32, (1, E), 1)
    neg_inf = jnp.float32(-jnp.inf)

    for img in range(IPS):
        planes = planes_ref[img]
        pee_b = planes[0]
        peo_b = planes[1]
        poe_b = planes[2]
        poo_b = planes[3]

        a_easy = jnp.concatenate([pee_b, peo_b, poe_b, poo_b], axis=0)
        a_hard = jnp.concatenate([
            shift_c(shift_r(poo_b)),
            shift_r(poe_b),
            shift_r(poo_b),
            shift_c(peo_b),
            shift_c(poo_b),
        ], axis=0)

        gx = gs_ref[0, img:img + 1, :] * np.float32(1.0 / (H * W))
        logits = jnp.dot(gx, wg_ref[...],
                         preferred_element_type=jnp.float32)

        m1 = jnp.max(logits, axis=1, keepdims=True)
        a1 = jnp.min(jnp.where(logits == m1, eidx, E), axis=1, keepdims=True)
        l2 = jnp.where(eidx == a1, neg_inf, logits)
        m2 = jnp.max(l2, axis=1, keepdims=True)
        a2 = jnp.min(jnp.where(l2 == m2, eidx, E), axis=1, keepdims=True)

        d = jnp.exp(m2 - m1)
        g1 = 1.0 / (1.0 + d)
        g2 = d / (1.0 + d)

        acc_ref[0:1, :] += (jnp.where(eidx == a1, g1, 0.0)
                            + jnp.where(eidx == a2, g2, 0.0))
        acc_ref[1:2, :] += ((eidx == a1).astype(jnp.float32)
                            + (eidx == a2).astype(jnp.float32))

        def conv_ln(e_scalar, gate):
            w = wmat_ref[pl.ds(e_scalar, 1)][0]
            y = (jnp.dot(w[:, :4 * C1], a_easy,
                         preferred_element_type=jnp.float32)
                 + jnp.dot(w[:, 4 * C1:], a_hard,
                           preferred_element_type=jnp.float32))
            y = y + b_ref[pl.ds(e_scalar, 1)][0][:, :1]
            u = y.mean(axis=0, keepdims=True)
            yc = y - u
            s2 = (yc * yc).mean(axis=0, keepdims=True)
            yn = yc * jax.lax.rsqrt(s2 + LN_EPS)
            yn = (g_ref[pl.ds(e_scalar, 1)][0][:, :1] * yn
                  + beta_ref[pl.ds(e_scalar, 1)][0][:, :1])
            return gate * yn

        out = conv_ln(a1[0, 0], g1)
        out += conv_ln(a2[0, 0], g2)
        out += conv_ln(E, jnp.float32(1.0))
        out_ref[img] = out

    @pl.when(i == nstep - 1)
    def _():
        def cv_sq(v):
            m = v.mean(axis=1, keepdims=True)
            var = ((v - m) ** 2).sum(axis=1, keepdims=True) / (E - 1)
            return var / (m * m + 1e-10)

        loss_ref[...] = (cv_sq(acc_ref[0:1, :]) + cv_sq(acc_ref[1:2, :])) * 1e-2


@jax.jit
def kernel(x, expert_conv_w, expert_conv_b, expert_ln_w, expert_ln_b,
           shared_conv_w, shared_conv_b, shared_ln_w, shared_ln_b, w_gate):
    n = x.shape[0]

    gsums = pl.pallas_call(
        _gsum_kernel,
        grid=(n // GB,),
        in_specs=[pl.BlockSpec((GB, C1, H * W), lambda i: (i, 0, 0))],
        out_specs=pl.BlockSpec((GB, C1), lambda i: (i, 0)),
        out_shape=jax.ShapeDtypeStruct((n, C1), jnp.float32),
    )(x.reshape(n, C1, H * W))

    xr = x.reshape(n, C1, OH, 2, OW, 2)
    planes = xr.transpose(0, 3, 5, 1, 2, 4).reshape(n, 4, C1, NPOS)
    planes = planes.astype(jnp.bfloat16)

    w_all = jnp.concatenate([expert_conv_w, shared_conv_w[None]], axis=0)
    w9 = w_all.transpose(0, 1, 3, 4, 2).reshape(E + 1, C2, KTAPS, C1)
    wmat = w9[:, :, jnp.array([4, 5, 7, 8, 0, 1, 2, 3, 6])].reshape(
        E + 1, C2, AROWS)
    wmat = wmat.astype(jnp.bfloat16)
    b_all = jnp.concatenate([expert_conv_b, shared_conv_b[None]], axis=0)
    g_all = jnp.concatenate([expert_ln_w, shared_ln_w[None]], axis=0)
    beta_all = jnp.concatenate([expert_ln_b, shared_ln_b[None]], axis=0)
    b_col = jnp.broadcast_to(b_all[:, :, None], (E + 1, C2, 128))
    g_col = jnp.broadcast_to(g_all[:, :, None], (E + 1, C2, 128))
    beta_col = jnp.broadcast_to(beta_all[:, :, None], (E + 1, C2, 128))
    cmask = (jnp.arange(NPOS, dtype=jnp.int32) % OW != 0)[None, :]
    cmask = cmask.astype(jnp.bfloat16)

    out, loss = pl.pallas_call(
        _moe_kernel,
        grid=(n // IPS,),
        in_specs=[
            pl.BlockSpec((1, IPS, C1), lambda i: (i, 0, 0)),
            pl.BlockSpec((IPS, 4, C1, NPOS), lambda i: (i, 0, 0, 0)),
            pl.BlockSpec((E + 1, C2, AROWS), lambda i: (0, 0, 0)),
            pl.BlockSpec((E + 1, C2, 128), lambda i: (0, 0, 0)),
            pl.BlockSpec((E + 1, C2, 128), lambda i: (0, 0, 0)),
            pl.BlockSpec((E + 1, C2, 128), lambda i: (0, 0, 0)),
            pl.BlockSpec((C1, E), lambda i: (0, 0)),
            pl.BlockSpec((1, NPOS), lambda i: (0, 0)),
        ],
        out_specs=[
            pl.BlockSpec((IPS, C2, NPOS), lambda i: (i, 0, 0)),
            pl.BlockSpec((1, 1), lambda i: (0, 0)),
        ],
        out_shape=[
            jax.ShapeDtypeStruct((n, C2, NPOS), jnp.float32),
            jax.ShapeDtypeStruct((1, 1), jnp.float32),
        ],
        scratch_shapes=[pltpu.VMEM((2, E), jnp.float32)],
    )(gsums.reshape(n // IPS, IPS, C1), planes, wmat, b_col, g_col,
      beta_col, w_gate, cmask)

    return out.reshape(n, C2, OH, OW), loss[0, 0]
